# Initial kernel scaffold; baseline (speedup 1.0000x reference)
import jax
import jax.numpy as jnp
from jax.experimental import pallas as pl


def kernel(edge_index, h, e, Eh0, Eh1, Eh2, Ee0, Ee1, W_proj, b_proj, W_e1, b_e1, W_e2, b_e2, b_conv, W_ih_gru, W_hh_gru, b_ih_gru, b_hh_gru, W_ih_lstm, W_hh_lstm, b_ih_lstm, b_hh_lstm, W_p1, b_p1, W_p2, b_p2):
    raise NotImplementedError("write your pallas kernel here")


# speedup vs baseline: 11.8796x; 11.8796x over previous
import functools

import jax
import jax.numpy as jnp
from jax import lax
from jax.experimental import pallas as pl
from jax.experimental.pallas import tpu as pltpu
from jax.experimental.pallas import tpu_sc as plsc

N = 10000
E = 320000
D = 16
NCLS = 64

NC = 2
NS = 16
NW = NC * NS
EPW = E // NW
CHUNK = 80
NCHUNK = EPW // CHUNK
N_PAD = 10240
NPT = N_PAD // NS

_f32 = jnp.float32


def _prep_body(h0, h1, h2, Eh0, Eh1, Eh2, Wp_a, Wp_b, Wp_c, b_proj,
               Ee0, Ee1, We1_a, We1_b, b_e1, We2t, be2r,
               x0_out, wall_out):
    lanes = lax.broadcasted_iota(jnp.int32, (N, 16), 1)
    oh0 = (h0[...] == lanes).astype(_f32)
    oh1 = (h1[...] == lanes).astype(_f32)
    oh2 = (h2[...] == lanes).astype(_f32)
    P0 = jnp.dot(Eh0[...], Wp_a[...], preferred_element_type=_f32)
    P1 = jnp.dot(Eh1[...], Wp_b[...], preferred_element_type=_f32)
    P2 = jnp.dot(Eh2[...], Wp_c[...], preferred_element_type=_f32)
    x0 = jnp.dot(oh0, P0, preferred_element_type=_f32)
    x0 = x0 + jnp.dot(oh1, P1, preferred_element_type=_f32)
    x0 = x0 + jnp.dot(oh2, P2, preferred_element_type=_f32)
    x0_out[...] = jnp.maximum(x0 + b_proj[...], 0.0)

    ii = lax.broadcasted_iota(jnp.int32, (NCLS, 8), 0)
    jj = lax.broadcasted_iota(jnp.int32, (NCLS, 8), 1)
    OH0 = ((ii // 8) == jj).astype(_f32)
    OH1 = ((ii % 8) == jj).astype(_f32)
    Q0 = jnp.dot(Ee0[...], We1_a[...], preferred_element_type=_f32)
    Q1 = jnp.dot(Ee1[...], We1_b[...], preferred_element_type=_f32)
    A = jnp.dot(OH0, Q0, preferred_element_type=_f32)
    A = A + jnp.dot(OH1, Q1, preferred_element_type=_f32)
    A = jnp.maximum(A + b_e1[...], 0.0)
    for d in range(D):
        wall_out[d, :, :] = (
            jnp.dot(A, We2t[d], preferred_element_type=_f32) + be2r[d][None, :])


def _prep(h0, h1, h2, Eh0, Eh1, Eh2, Wp_a, Wp_b, Wp_c, b_proj,
          Ee0, Ee1, We1_a, We1_b, b_e1, We2t, be2r):
    return pl.pallas_call(
        _prep_body,
        out_shape=(jax.ShapeDtypeStruct((N, D), _f32),
                   jax.ShapeDtypeStruct((D, NCLS, D), _f32)),
    )(h0, h1, h2, Eh0, Eh1, Eh2, Wp_a, Wp_b, Wp_c, b_proj,
      Ee0, Ee1, We1_a, We1_b, b_e1, We2t, be2r)


_RB = 2000


def _ymm_body(x_ref, w_ref, y_ref):
    y_ref[...] = jnp.dot(x_ref[...], w_ref[...], preferred_element_type=_f32)


def _ymm(x, w2d):
    return pl.pallas_call(
        _ymm_body,
        grid=(N // _RB,),
        in_specs=[pl.BlockSpec((_RB, D), lambda i: (i, 0)),
                  pl.BlockSpec((D, NCLS * D), lambda i: (0, 0))],
        out_specs=pl.BlockSpec((_RB, NCLS * D), lambda i: (i, 0)),
        out_shape=jax.ShapeDtypeStruct((N, NCLS * D), _f32),
    )(x, w2d)


def _sc_body(y_hbm, src_hbm, e0_hbm, e1_hbm, dst_hbm, out_hbm,
             src_v, e0_v, e1_v, dst_v, gidx_v, rows_v, zero_v, agg_sh, sem):
    cid = lax.axis_index("c")
    sid = lax.axis_index("s")
    wid = sid * NC + cid

    pltpu.sync_copy(src_hbm.at[wid], src_v)
    pltpu.sync_copy(e0_hbm.at[wid], e0_v)
    pltpu.sync_copy(e1_hbm.at[wid], e1_v)
    pltpu.sync_copy(dst_hbm.at[wid], dst_v)

    def zbody(i, c):
        zero_v[i, :] = jnp.zeros((16,), _f32)
        return c
    lax.fori_loop(0, NPT, zbody, 0)
    pltpu.sync_copy(zero_v, agg_sh.at[pl.ds(sid * NPT, NPT)])

    def gbody(j, c):
        for i in range(CHUNK // 16):
            sl = pl.ds(i * 16, 16)
            gidx_v[j, sl] = src_v[j, sl] * NCLS + e0_v[j, sl] * 8 + e1_v[j, sl]
        return c
    lax.fori_loop(0, NCHUNK, gbody, 0)

    plsc.subcore_barrier()

    def cbody(j, c):
        pltpu.async_copy(y_hbm.at[gidx_v.at[j]], rows_v, sem).wait()
        pltpu.sync_copy(rows_v, agg_sh.at[dst_v.at[j]], add=True)
        return c
    lax.fori_loop(0, NCHUNK, cbody, 0)

    plsc.subcore_barrier()

    pltpu.sync_copy(agg_sh.at[pl.ds(sid * NPT, NPT)],
                    out_hbm.at[pl.ds(cid * N_PAD + sid * NPT, NPT)])


def _sc_step(y2d, srcb, e0b, e1b, dstb):
    mesh = plsc.VectorSubcoreMesh(core_axis_name="c", subcore_axis_name="s")
    k = functools.partial(
        pl.kernel,
        mesh=mesh,
        out_type=jax.ShapeDtypeStruct((NC * N_PAD, D), _f32),
        scratch_types=[
            pltpu.VMEM((NCHUNK, CHUNK), jnp.int32),
            pltpu.VMEM((NCHUNK, CHUNK), jnp.int32),
            pltpu.VMEM((NCHUNK, CHUNK), jnp.int32),
            pltpu.VMEM((NCHUNK, CHUNK), jnp.int32),
            pltpu.VMEM((NCHUNK, CHUNK), jnp.int32),
            pltpu.VMEM((CHUNK, D), _f32),
            pltpu.VMEM((NPT, D), _f32),
            pltpu.VMEM_SHARED((N_PAD, D), _f32),
            pltpu.SemaphoreType.DMA,
        ],
        compiler_params=pltpu.CompilerParams(use_tc_tiling_on_sc=False),
    )(_sc_body)
    return k(y2d, srcb, e0b, e1b, dstb)


def _gru_math(agg2_ref, hid_ref, b_conv,
              Wri, Wzi, Wni, Wrh, Wzh, Wnh, bri, bzi, bni, brh, bzh, bnh):
    agg = agg2_ref[0] + agg2_ref[1] + b_conv[...]
    x_in = jnp.maximum(agg, 0.0)
    hid = hid_ref[...]

    def mm(a, b):
        return jnp.dot(a, b[...], preferred_element_type=_f32)

    r = jax.nn.sigmoid(mm(x_in, Wri) + bri[...] + mm(hid, Wrh) + brh[...])
    z = jax.nn.sigmoid(mm(x_in, Wzi) + bzi[...] + mm(hid, Wzh) + bzh[...])
    n = jnp.tanh(mm(x_in, Wni) + bni[...] + r * (mm(hid, Wnh) + bnh[...]))
    return (1.0 - z) * n + z * hid


def _gru_y_body(agg2_ref, hid_ref, b_conv, Wri, Wzi, Wni, Wrh, Wzh, Wnh,
                bri, bzi, bni, brh, bzh, bnh, w2d_ref, hnew_ref, y_ref):
    hnew = _gru_math(agg2_ref, hid_ref, b_conv, Wri, Wzi, Wni, Wrh, Wzh, Wnh,
                     bri, bzi, bni, brh, bzh, bnh)
    hnew_ref[...] = hnew
    y_ref[...] = jnp.dot(hnew, w2d_ref[...], preferred_element_type=_f32)


def _gru_last_body(agg2_ref, hid_ref, b_conv, Wri, Wzi, Wni, Wrh, Wzh, Wnh,
                   bri, bzi, bni, brh, bzh, bnh, hnew_ref):
    hnew_ref[...] = _gru_math(agg2_ref, hid_ref, b_conv, Wri, Wzi, Wni,
                              Wrh, Wzh, Wnh, bri, bzi, bni, brh, bzh, bnh)


def _gru_step(agg2, hid, w2d, b_conv2d, gw, with_y):
    wspec = [pl.BlockSpec((D, D), lambda i: (0, 0))] * 6
    bspec = [pl.BlockSpec((1, D), lambda i: (0, 0))] * 6
    in_specs = ([pl.BlockSpec((NC, _RB, D), lambda i: (0, i, 0)),
                 pl.BlockSpec((_RB, D), lambda i: (i, 0)),
                 pl.BlockSpec((1, D), lambda i: (0, 0))]
                + wspec + bspec)
    args = [agg2, hid, b_conv2d] + gw
    if with_y:
        in_specs.append(pl.BlockSpec((D, NCLS * D), lambda i: (0, 0)))
        return pl.pallas_call(
            _gru_y_body,
            grid=(N // _RB,),
            in_specs=in_specs,
            out_specs=(pl.BlockSpec((_RB, D), lambda i: (i, 0)),
                       pl.BlockSpec((_RB, NCLS * D), lambda i: (i, 0))),
            out_shape=(jax.ShapeDtypeStruct((N, D), _f32),
                       jax.ShapeDtypeStruct((N, NCLS * D), _f32)),
        )(*args, w2d)
    return pl.pallas_call(
        _gru_last_body,
        grid=(N // _RB,),
        in_specs=in_specs,
        out_specs=pl.BlockSpec((_RB, D), lambda i: (i, 0)),
        out_shape=jax.ShapeDtypeStruct((N, D), _f32),
    )(*args)


def _tail_body(x_ref, Wi_q, Wi_r, Wf_q, Wf_r, Wg_q, Wg_r, Wo_q, Wo_r,
               Whi, Whf, Whg, Who, bi, bf, bg, bo,
               Wp1q, Wp1r, bp1, Wp2, bp2, out_ref):
    x = x_ref[...]

    def mm(a, b):
        return jnp.dot(a, b[...], preferred_element_type=_f32)

    hc = jnp.zeros((1, D), _f32)
    cc = jnp.zeros((1, D), _f32)
    q = jnp.zeros((1, D), _f32)
    readout = jnp.zeros((1, D), _f32)
    for _ in range(3):
        i_g = jax.nn.sigmoid(mm(q, Wi_q) + mm(readout, Wi_r) + mm(hc, Whi)
                             + bi[...])
        f_g = jax.nn.sigmoid(mm(q, Wf_q) + mm(readout, Wf_r) + mm(hc, Whf)
                             + bf[...])
        g_g = jnp.tanh(mm(q, Wg_q) + mm(readout, Wg_r) + mm(hc, Whg)
                       + bg[...])
        o_g = jax.nn.sigmoid(mm(q, Wo_q) + mm(readout, Wo_r) + mm(hc, Who)
                             + bo[...])
        cc = f_g * cc + i_g * g_g
        hc = o_g * jnp.tanh(cc)
        q = hc
        en = jnp.sum(x * q, axis=-1, keepdims=True)
        mx = jnp.max(en)
        w = jnp.exp(en - mx)
        readout = jnp.sum(x * w, axis=0, keepdims=True) / jnp.sum(w)

    hmlp = jnp.maximum(mm(q, Wp1q) + mm(readout, Wp1r) + bp1[...], 0.0)
    out_ref[...] = mm(hmlp, Wp2) + bp2[...]


def _tail(x, tw):
    return pl.pallas_call(
        _tail_body,
        out_shape=jax.ShapeDtypeStruct((1, D), _f32),
    )(x, *tw)


def kernel(edge_index, h, e, Eh0, Eh1, Eh2, Ee0, Ee1, W_proj, b_proj,
           W_e1, b_e1, W_e2, b_e2, b_conv,
           W_ih_gru, W_hh_gru, b_ih_gru, b_hh_gru,
           W_ih_lstm, W_hh_lstm, b_ih_lstm, b_hh_lstm,
           W_p1, b_p1, W_p2, b_p2):
    srcb = edge_index[0].reshape(NW, NCHUNK, CHUNK).astype(jnp.int32)
    dstb = edge_index[1].reshape(NW, NCHUNK, CHUNK).astype(jnp.int32)
    e0b = e[:, 0].reshape(NW, NCHUNK, CHUNK).astype(jnp.int32)
    e1b = e[:, 1].reshape(NW, NCHUNK, CHUNK).astype(jnp.int32)
    h0 = h[:, 0:1].astype(jnp.int32)
    h1 = h[:, 1:2].astype(jnp.int32)
    h2 = h[:, 2:3].astype(jnp.int32)
    We2t = W_e2.reshape(2 * D, D, D).transpose(1, 0, 2)
    be2r = b_e2.reshape(D, D)
    Wgi = W_ih_gru.T
    Wgh = W_hh_gru.T
    gw = [Wgi[:, 0:D], Wgi[:, D:2 * D], Wgi[:, 2 * D:3 * D],
          Wgh[:, 0:D], Wgh[:, D:2 * D], Wgh[:, 2 * D:3 * D],
          b_ih_gru[0:D].reshape(1, D), b_ih_gru[D:2 * D].reshape(1, D),
          b_ih_gru[2 * D:3 * D].reshape(1, D),
          b_hh_gru[0:D].reshape(1, D), b_hh_gru[D:2 * D].reshape(1, D),
          b_hh_gru[2 * D:3 * D].reshape(1, D)]
    Wli = W_ih_lstm.T
    Wlh = W_hh_lstm.T
    bl_i = b_ih_lstm.reshape(1, 4 * D)
    bl_h = b_hh_lstm.reshape(1, 4 * D)
    tw = []
    for g in range(4):
        tw += [Wli[0:D, g * D:(g + 1) * D], Wli[D:2 * D, g * D:(g + 1) * D]]
    tw += [Wlh[:, g * D:(g + 1) * D] for g in range(4)]
    tw += [(bl_i[:, g * D:(g + 1) * D] + bl_h[:, g * D:(g + 1) * D])
           for g in range(4)]
    tw += [W_p1[0:D], W_p1[D:2 * D], b_p1.reshape(1, D), W_p2,
           b_p2.reshape(1, D)]

    x0, wall = _prep(h0, h1, h2, Eh0, Eh1, Eh2,
                     W_proj[0:8], W_proj[8:12], W_proj[12:16],
                     b_proj.reshape(1, D), Ee0, Ee1,
                     W_e1[0:4], W_e1[4:8], b_e1.reshape(1, 2 * D), We2t, be2r)
    w2d = wall.reshape(D, NCLS * D)
    y = _ymm(x0, w2d)
    hid = x0
    for t in range(3):
        agg2 = _sc_step(y.reshape(N * NCLS, D), srcb, e0b, e1b, dstb)
        agg2 = agg2.reshape(NC, N_PAD, D)[:, :N, :]
        if t < 2:
            hid, y = _gru_step(agg2, hid, w2d, b_conv.reshape(1, D), gw,
                               with_y=True)
        else:
            hid = _gru_step(agg2, hid, None, b_conv.reshape(1, D), gw,
                            with_y=False)
    return _tail(hid, tw)


# speedup vs baseline: 15.5625x; 1.3100x over previous
import functools

import jax
import jax.numpy as jnp
from jax import lax
from jax.experimental import pallas as pl
from jax.experimental.pallas import tpu as pltpu
from jax.experimental.pallas import tpu_sc as plsc

N = 10000
E = 320000
D = 16
NCLS = 64

NC = 2
NS = 16
NW = NC * NS
EPW = E // NW
CHUNK = 80
NCHUNK = EPW // CHUNK
N_PAD = 10240
NPT = N_PAD // NS

_f32 = jnp.float32


def _prep_body(h0, h1, h2, Eh0, Eh1, Eh2, Wp_a, Wp_b, Wp_c, b_proj,
               Ee0, Ee1, We1_a, We1_b, b_e1, We2t, be2r,
               x0_out, wall_out):
    lanes = lax.broadcasted_iota(jnp.int32, (N, 16), 1)
    oh0 = (h0[...] == lanes).astype(_f32)
    oh1 = (h1[...] == lanes).astype(_f32)
    oh2 = (h2[...] == lanes).astype(_f32)
    P0 = jnp.dot(Eh0[...], Wp_a[...], preferred_element_type=_f32)
    P1 = jnp.dot(Eh1[...], Wp_b[...], preferred_element_type=_f32)
    P2 = jnp.dot(Eh2[...], Wp_c[...], preferred_element_type=_f32)
    x0 = jnp.dot(oh0, P0, preferred_element_type=_f32)
    x0 = x0 + jnp.dot(oh1, P1, preferred_element_type=_f32)
    x0 = x0 + jnp.dot(oh2, P2, preferred_element_type=_f32)
    x0_out[...] = jnp.maximum(x0 + b_proj[...], 0.0)

    ii = lax.broadcasted_iota(jnp.int32, (NCLS, 8), 0)
    jj = lax.broadcasted_iota(jnp.int32, (NCLS, 8), 1)
    OH0 = ((ii // 8) == jj).astype(_f32)
    OH1 = ((ii % 8) == jj).astype(_f32)
    Q0 = jnp.dot(Ee0[...], We1_a[...], preferred_element_type=_f32)
    Q1 = jnp.dot(Ee1[...], We1_b[...], preferred_element_type=_f32)
    A = jnp.dot(OH0, Q0, preferred_element_type=_f32)
    A = A + jnp.dot(OH1, Q1, preferred_element_type=_f32)
    A = jnp.maximum(A + b_e1[...], 0.0)
    for d in range(D):
        wall_out[d, :, :] = (
            jnp.dot(A, We2t[d], preferred_element_type=_f32) + be2r[d][None, :])


def _prep(h0, h1, h2, Eh0, Eh1, Eh2, Wp_a, Wp_b, Wp_c, b_proj,
          Ee0, Ee1, We1_a, We1_b, b_e1, We2t, be2r):
    return pl.pallas_call(
        _prep_body,
        out_shape=(jax.ShapeDtypeStruct((N, D), _f32),
                   jax.ShapeDtypeStruct((D, NCLS, D), _f32)),
    )(h0, h1, h2, Eh0, Eh1, Eh2, Wp_a, Wp_b, Wp_c, b_proj,
      Ee0, Ee1, We1_a, We1_b, b_e1, We2t, be2r)


_RB = 2000


def _ymm_body(x_ref, w_ref, y_ref):
    y_ref[...] = jnp.dot(x_ref[...], w_ref[...], preferred_element_type=_f32)


def _ymm(x, w2d):
    return pl.pallas_call(
        _ymm_body,
        grid=(N // _RB,),
        in_specs=[pl.BlockSpec((_RB, D), lambda i: (i, 0)),
                  pl.BlockSpec((D, NCLS * D), lambda i: (0, 0))],
        out_specs=pl.BlockSpec((_RB, NCLS * D), lambda i: (i, 0)),
        out_shape=jax.ShapeDtypeStruct((N, NCLS * D), _f32),
    )(x, w2d)


def _sc_body(y_hbm, src_hbm, e0_hbm, e1_hbm, dst_hbm, out_hbm,
             src_v, e0_v, e1_v, dst_v, gidx_v, rows_v, rows2_v, zero_v,
             agg_sh, sem, sem2):
    cid = lax.axis_index("c")
    sid = lax.axis_index("s")
    wid = sid * NC + cid

    pltpu.sync_copy(src_hbm.at[wid], src_v)
    pltpu.sync_copy(e0_hbm.at[wid], e0_v)
    pltpu.sync_copy(e1_hbm.at[wid], e1_v)
    pltpu.sync_copy(dst_hbm.at[wid], dst_v)

    def zbody(i, c):
        zero_v[i, :] = jnp.zeros((16,), _f32)
        return c
    lax.fori_loop(0, NPT, zbody, 0)
    pltpu.sync_copy(zero_v, agg_sh.at[pl.ds(sid * NPT, NPT)])

    def gbody(j, c):
        for i in range(CHUNK // 16):
            sl = pl.ds(i * 16, 16)
            gidx_v[j, sl] = src_v[j, sl] * NCLS + e0_v[j, sl] * 8 + e1_v[j, sl]
        return c
    lax.fori_loop(0, NCHUNK, gbody, 0)

    plsc.subcore_barrier()

    rows = (rows_v, rows2_v)
    sems = (sem, sem2)
    pltpu.async_copy(y_hbm.at[gidx_v.at[0]], rows[0], sems[0])
    pltpu.async_copy(y_hbm.at[gidx_v.at[1]], rows[1], sems[1])

    def cbody(i, c):
        j0 = i * 2
        for b in range(2):
            j = j0 + b
            pltpu.make_async_copy(y_hbm.at[gidx_v.at[j]], rows[b],
                                  sems[b]).wait()
            pltpu.sync_copy(rows[b], agg_sh.at[dst_v.at[j]], add=True)

            @pl.when(j + 2 < NCHUNK)
            def _():
                pltpu.async_copy(y_hbm.at[gidx_v.at[j + 2]], rows[b], sems[b])
        return c
    lax.fori_loop(0, (NCHUNK - 1) // 2, cbody, 0)
    j = NCHUNK - 1
    pltpu.make_async_copy(y_hbm.at[gidx_v.at[j]], rows[0], sems[0]).wait()
    pltpu.sync_copy(rows[0], agg_sh.at[dst_v.at[j]], add=True)

    plsc.subcore_barrier()

    pltpu.sync_copy(agg_sh.at[pl.ds(sid * NPT, NPT)],
                    out_hbm.at[pl.ds(cid * N_PAD + sid * NPT, NPT)])


def _sc_step(y2d, srcb, e0b, e1b, dstb):
    mesh = plsc.VectorSubcoreMesh(core_axis_name="c", subcore_axis_name="s")
    k = functools.partial(
        pl.kernel,
        mesh=mesh,
        out_type=jax.ShapeDtypeStruct((NC * N_PAD, D), _f32),
        scratch_types=[
            pltpu.VMEM((NCHUNK, CHUNK), jnp.int32),
            pltpu.VMEM((NCHUNK, CHUNK), jnp.int32),
            pltpu.VMEM((NCHUNK, CHUNK), jnp.int32),
            pltpu.VMEM((NCHUNK, CHUNK), jnp.int32),
            pltpu.VMEM((NCHUNK, CHUNK), jnp.int32),
            pltpu.VMEM((CHUNK, D), _f32),
            pltpu.VMEM((CHUNK, D), _f32),
            pltpu.VMEM((NPT, D), _f32),
            pltpu.VMEM_SHARED((N_PAD, D), _f32),
            pltpu.SemaphoreType.DMA,
            pltpu.SemaphoreType.DMA,
        ],
        compiler_params=pltpu.CompilerParams(use_tc_tiling_on_sc=False),
    )(_sc_body)
    return k(y2d, srcb, e0b, e1b, dstb)


def _gru_math(agg2_ref, hid_ref, b_conv,
              Wri, Wzi, Wni, Wrh, Wzh, Wnh, bri, bzi, bni, brh, bzh, bnh):
    agg = agg2_ref[0] + agg2_ref[1] + b_conv[...]
    x_in = jnp.maximum(agg, 0.0)
    hid = hid_ref[...]

    def mm(a, b):
        return jnp.dot(a, b[...], preferred_element_type=_f32)

    r = jax.nn.sigmoid(mm(x_in, Wri) + bri[...] + mm(hid, Wrh) + brh[...])
    z = jax.nn.sigmoid(mm(x_in, Wzi) + bzi[...] + mm(hid, Wzh) + bzh[...])
    n = jnp.tanh(mm(x_in, Wni) + bni[...] + r * (mm(hid, Wnh) + bnh[...]))
    return (1.0 - z) * n + z * hid


def _gru_y_body(agg2_ref, hid_ref, b_conv, Wri, Wzi, Wni, Wrh, Wzh, Wnh,
                bri, bzi, bni, brh, bzh, bnh, w2d_ref, hnew_ref, y_ref):
    hnew = _gru_math(agg2_ref, hid_ref, b_conv, Wri, Wzi, Wni, Wrh, Wzh, Wnh,
                     bri, bzi, bni, brh, bzh, bnh)
    hnew_ref[...] = hnew
    y_ref[...] = jnp.dot(hnew, w2d_ref[...], preferred_element_type=_f32)


def _gru_last_body(agg2_ref, hid_ref, b_conv, Wri, Wzi, Wni, Wrh, Wzh, Wnh,
                   bri, bzi, bni, brh, bzh, bnh, hnew_ref):
    hnew_ref[...] = _gru_math(agg2_ref, hid_ref, b_conv, Wri, Wzi, Wni,
                              Wrh, Wzh, Wnh, bri, bzi, bni, brh, bzh, bnh)


def _gru_step(agg2, hid, w2d, b_conv2d, gw, with_y):
    wspec = [pl.BlockSpec((D, D), lambda i: (0, 0))] * 6
    bspec = [pl.BlockSpec((1, D), lambda i: (0, 0))] * 6
    in_specs = ([pl.BlockSpec((NC, _RB, D), lambda i: (0, i, 0)),
                 pl.BlockSpec((_RB, D), lambda i: (i, 0)),
                 pl.BlockSpec((1, D), lambda i: (0, 0))]
                + wspec + bspec)
    args = [agg2, hid, b_conv2d] + gw
    if with_y:
        in_specs.append(pl.BlockSpec((D, NCLS * D), lambda i: (0, 0)))
        return pl.pallas_call(
            _gru_y_body,
            grid=(N // _RB,),
            in_specs=in_specs,
            out_specs=(pl.BlockSpec((_RB, D), lambda i: (i, 0)),
                       pl.BlockSpec((_RB, NCLS * D), lambda i: (i, 0))),
            out_shape=(jax.ShapeDtypeStruct((N, D), _f32),
                       jax.ShapeDtypeStruct((N, NCLS * D), _f32)),
        )(*args, w2d)
    return pl.pallas_call(
        _gru_last_body,
        grid=(N // _RB,),
        in_specs=in_specs,
        out_specs=pl.BlockSpec((_RB, D), lambda i: (i, 0)),
        out_shape=jax.ShapeDtypeStruct((N, D), _f32),
    )(*args)


def _tail_body(x_ref, Wi_q, Wi_r, Wf_q, Wf_r, Wg_q, Wg_r, Wo_q, Wo_r,
               Whi, Whf, Whg, Who, bi, bf, bg, bo,
               Wp1q, Wp1r, bp1, Wp2, bp2, out_ref):
    x = x_ref[...]

    def mm(a, b):
        return jnp.dot(a, b[...], preferred_element_type=_f32)

    hc = jnp.zeros((1, D), _f32)
    cc = jnp.zeros((1, D), _f32)
    q = jnp.zeros((1, D), _f32)
    readout = jnp.zeros((1, D), _f32)
    for _ in range(3):
        i_g = jax.nn.sigmoid(mm(q, Wi_q) + mm(readout, Wi_r) + mm(hc, Whi)
                             + bi[...])
        f_g = jax.nn.sigmoid(mm(q, Wf_q) + mm(readout, Wf_r) + mm(hc, Whf)
                             + bf[...])
        g_g = jnp.tanh(mm(q, Wg_q) + mm(readout, Wg_r) + mm(hc, Whg)
                       + bg[...])
        o_g = jax.nn.sigmoid(mm(q, Wo_q) + mm(readout, Wo_r) + mm(hc, Who)
                             + bo[...])
        cc = f_g * cc + i_g * g_g
        hc = o_g * jnp.tanh(cc)
        q = hc
        en = jnp.sum(x * q, axis=-1, keepdims=True)
        mx = jnp.max(en)
        w = jnp.exp(en - mx)
        readout = jnp.sum(x * w, axis=0, keepdims=True) / jnp.sum(w)

    hmlp = jnp.maximum(mm(q, Wp1q) + mm(readout, Wp1r) + bp1[...], 0.0)
    out_ref[...] = mm(hmlp, Wp2) + bp2[...]


def _tail(x, tw):
    return pl.pallas_call(
        _tail_body,
        out_shape=jax.ShapeDtypeStruct((1, D), _f32),
    )(x, *tw)


def kernel(edge_index, h, e, Eh0, Eh1, Eh2, Ee0, Ee1, W_proj, b_proj,
           W_e1, b_e1, W_e2, b_e2, b_conv,
           W_ih_gru, W_hh_gru, b_ih_gru, b_hh_gru,
           W_ih_lstm, W_hh_lstm, b_ih_lstm, b_hh_lstm,
           W_p1, b_p1, W_p2, b_p2):
    srcb = edge_index[0].reshape(NW, NCHUNK, CHUNK).astype(jnp.int32)
    dstb = edge_index[1].reshape(NW, NCHUNK, CHUNK).astype(jnp.int32)
    e0b = e[:, 0].reshape(NW, NCHUNK, CHUNK).astype(jnp.int32)
    e1b = e[:, 1].reshape(NW, NCHUNK, CHUNK).astype(jnp.int32)
    h0 = h[:, 0:1].astype(jnp.int32)
    h1 = h[:, 1:2].astype(jnp.int32)
    h2 = h[:, 2:3].astype(jnp.int32)
    We2t = W_e2.reshape(2 * D, D, D).transpose(1, 0, 2)
    be2r = b_e2.reshape(D, D)
    Wgi = W_ih_gru.T
    Wgh = W_hh_gru.T
    gw = [Wgi[:, 0:D], Wgi[:, D:2 * D], Wgi[:, 2 * D:3 * D],
          Wgh[:, 0:D], Wgh[:, D:2 * D], Wgh[:, 2 * D:3 * D],
          b_ih_gru[0:D].reshape(1, D), b_ih_gru[D:2 * D].reshape(1, D),
          b_ih_gru[2 * D:3 * D].reshape(1, D),
          b_hh_gru[0:D].reshape(1, D), b_hh_gru[D:2 * D].reshape(1, D),
          b_hh_gru[2 * D:3 * D].reshape(1, D)]
    Wli = W_ih_lstm.T
    Wlh = W_hh_lstm.T
    bl_i = b_ih_lstm.reshape(1, 4 * D)
    bl_h = b_hh_lstm.reshape(1, 4 * D)
    tw = []
    for g in range(4):
        tw += [Wli[0:D, g * D:(g + 1) * D], Wli[D:2 * D, g * D:(g + 1) * D]]
    tw += [Wlh[:, g * D:(g + 1) * D] for g in range(4)]
    tw += [(bl_i[:, g * D:(g + 1) * D] + bl_h[:, g * D:(g + 1) * D])
           for g in range(4)]
    tw += [W_p1[0:D], W_p1[D:2 * D], b_p1.reshape(1, D), W_p2,
           b_p2.reshape(1, D)]

    x0, wall = _prep(h0, h1, h2, Eh0, Eh1, Eh2,
                     W_proj[0:8], W_proj[8:12], W_proj[12:16],
                     b_proj.reshape(1, D), Ee0, Ee1,
                     W_e1[0:4], W_e1[4:8], b_e1.reshape(1, 2 * D), We2t, be2r)
    w2d = wall.reshape(D, NCLS * D)
    y = _ymm(x0, w2d)
    hid = x0
    for t in range(3):
        agg2 = _sc_step(y.reshape(N * NCLS, D), srcb, e0b, e1b, dstb)
        agg2 = agg2.reshape(NC, N_PAD, D)[:, :N, :]
        if t < 2:
            hid, y = _gru_step(agg2, hid, w2d, b_conv.reshape(1, D), gw,
                               with_y=True)
        else:
            hid = _gru_step(agg2, hid, None, b_conv.reshape(1, D), gw,
                            with_y=False)
    return _tail(hid, tw)


# speedup vs baseline: 18.3583x; 1.1796x over previous
import functools

import jax
import jax.numpy as jnp
from jax import lax
from jax.experimental import pallas as pl
from jax.experimental.pallas import tpu as pltpu
from jax.experimental.pallas import tpu_sc as plsc

N = 10000
E = 320000
D = 16
NCLS = 64

NC = 2
NS = 16
NW = NC * NS
EPW = E // NW
CHUNK = 80
NCHUNK = EPW // CHUNK
N_PAD = 10240
NPT = N_PAD // NS

_f32 = jnp.float32


def _prep_body(h0, h1, h2, Eh0, Eh1, Eh2, Wp_a, Wp_b, Wp_c, b_proj,
               Ee0, Ee1, We1_a, We1_b, b_e1, We2t, be2r,
               x0_out, wall_out):
    lanes = lax.broadcasted_iota(jnp.int32, (N, 16), 1)
    oh0 = (h0[...] == lanes).astype(_f32)
    oh1 = (h1[...] == lanes).astype(_f32)
    oh2 = (h2[...] == lanes).astype(_f32)
    P0 = jnp.dot(Eh0[...], Wp_a[...], preferred_element_type=_f32)
    P1 = jnp.dot(Eh1[...], Wp_b[...], preferred_element_type=_f32)
    P2 = jnp.dot(Eh2[...], Wp_c[...], preferred_element_type=_f32)
    x0 = jnp.dot(oh0, P0, preferred_element_type=_f32)
    x0 = x0 + jnp.dot(oh1, P1, preferred_element_type=_f32)
    x0 = x0 + jnp.dot(oh2, P2, preferred_element_type=_f32)
    x0_out[...] = jnp.maximum(x0 + b_proj[...], 0.0)

    ii = lax.broadcasted_iota(jnp.int32, (NCLS, 8), 0)
    jj = lax.broadcasted_iota(jnp.int32, (NCLS, 8), 1)
    OH0 = ((ii // 8) == jj).astype(_f32)
    OH1 = ((ii % 8) == jj).astype(_f32)
    Q0 = jnp.dot(Ee0[...], We1_a[...], preferred_element_type=_f32)
    Q1 = jnp.dot(Ee1[...], We1_b[...], preferred_element_type=_f32)
    A = jnp.dot(OH0, Q0, preferred_element_type=_f32)
    A = A + jnp.dot(OH1, Q1, preferred_element_type=_f32)
    A = jnp.maximum(A + b_e1[...], 0.0)
    for d in range(D):
        wall_out[d, :, :] = (
            jnp.dot(A, We2t[d], preferred_element_type=_f32) + be2r[d][None, :])


def _prep(h0, h1, h2, Eh0, Eh1, Eh2, Wp_a, Wp_b, Wp_c, b_proj,
          Ee0, Ee1, We1_a, We1_b, b_e1, We2t, be2r):
    return pl.pallas_call(
        _prep_body,
        out_shape=(jax.ShapeDtypeStruct((N, D), _f32),
                   jax.ShapeDtypeStruct((D, NCLS, D), _f32)),
    )(h0, h1, h2, Eh0, Eh1, Eh2, Wp_a, Wp_b, Wp_c, b_proj,
      Ee0, Ee1, We1_a, We1_b, b_e1, We2t, be2r)


_RB = 2000


def _ymm_body(x_ref, w_ref, y_ref):
    y_ref[...] = jnp.dot(x_ref[...], w_ref[...], preferred_element_type=_f32)


def _ymm(x, w2d):
    return pl.pallas_call(
        _ymm_body,
        grid=(N // _RB,),
        in_specs=[pl.BlockSpec((_RB, D), lambda i: (i, 0)),
                  pl.BlockSpec((D, NCLS * D), lambda i: (0, 0))],
        out_specs=pl.BlockSpec((_RB, NCLS * D), lambda i: (i, 0)),
        out_shape=jax.ShapeDtypeStruct((N, NCLS * D), _f32),
    )(x, w2d)


NBUF = 4


def _sc_body(y_hbm, src_hbm, e0_hbm, e1_hbm, dst_hbm, out_hbm,
             src_v, e0_v, e1_v, dst_v, gidx_v, rows_v, rows2_v, rows3_v,
             rows4_v, zero_v, agg_sh, sem, sem2, sem3, sem4):
    cid = lax.axis_index("c")
    sid = lax.axis_index("s")
    wid = sid * NC + cid

    pltpu.sync_copy(src_hbm.at[wid], src_v)
    pltpu.sync_copy(e0_hbm.at[wid], e0_v)
    pltpu.sync_copy(e1_hbm.at[wid], e1_v)
    pltpu.sync_copy(dst_hbm.at[wid], dst_v)

    def zbody(i, c):
        zero_v[i, :] = jnp.zeros((16,), _f32)
        return c
    lax.fori_loop(0, NPT, zbody, 0)
    pltpu.sync_copy(zero_v, agg_sh.at[pl.ds(sid * NPT, NPT)])

    def gbody(j, c):
        for i in range(CHUNK // 16):
            sl = pl.ds(i * 16, 16)
            gidx_v[j, sl] = src_v[j, sl] * NCLS + e0_v[j, sl] * 8 + e1_v[j, sl]
        return c
    lax.fori_loop(0, NCHUNK, gbody, 0)

    plsc.subcore_barrier()

    rows = (rows_v, rows2_v, rows3_v, rows4_v)
    sems = (sem, sem2, sem3, sem4)
    for b in range(NBUF):
        pltpu.async_copy(y_hbm.at[gidx_v.at[b]], rows[b], sems[b])

    def cbody(i, c):
        j0 = i * NBUF
        for b in range(NBUF):
            j = j0 + b
            pltpu.make_async_copy(y_hbm.at[gidx_v.at[j]], rows[b],
                                  sems[b]).wait()
            pltpu.sync_copy(rows[b], agg_sh.at[dst_v.at[j]], add=True)

            @pl.when(j + NBUF < NCHUNK)
            def _():
                pltpu.async_copy(y_hbm.at[gidx_v.at[j + NBUF]], rows[b],
                                 sems[b])
        return c
    lax.fori_loop(0, NCHUNK // NBUF, cbody, 0)
    j = NCHUNK - 1
    pltpu.make_async_copy(y_hbm.at[gidx_v.at[j]], rows[0], sems[0]).wait()
    pltpu.sync_copy(rows[0], agg_sh.at[dst_v.at[j]], add=True)

    plsc.subcore_barrier()

    pltpu.sync_copy(agg_sh.at[pl.ds(sid * NPT, NPT)],
                    out_hbm.at[pl.ds(cid * N_PAD + sid * NPT, NPT)])


def _sc_step(y2d, srcb, e0b, e1b, dstb):
    mesh = plsc.VectorSubcoreMesh(core_axis_name="c", subcore_axis_name="s")
    k = functools.partial(
        pl.kernel,
        mesh=mesh,
        out_type=jax.ShapeDtypeStruct((NC * N_PAD, D), _f32),
        scratch_types=[
            pltpu.VMEM((NCHUNK, CHUNK), jnp.int32),
            pltpu.VMEM((NCHUNK, CHUNK), jnp.int32),
            pltpu.VMEM((NCHUNK, CHUNK), jnp.int32),
            pltpu.VMEM((NCHUNK, CHUNK), jnp.int32),
            pltpu.VMEM((NCHUNK, CHUNK), jnp.int32),
            pltpu.VMEM((CHUNK, D), _f32),
            pltpu.VMEM((CHUNK, D), _f32),
            pltpu.VMEM((CHUNK, D), _f32),
            pltpu.VMEM((CHUNK, D), _f32),
            pltpu.VMEM((NPT, D), _f32),
            pltpu.VMEM_SHARED((N_PAD, D), _f32),
            pltpu.SemaphoreType.DMA,
            pltpu.SemaphoreType.DMA,
            pltpu.SemaphoreType.DMA,
            pltpu.SemaphoreType.DMA,
        ],
        compiler_params=pltpu.CompilerParams(use_tc_tiling_on_sc=False),
    )(_sc_body)
    return k(y2d, srcb, e0b, e1b, dstb)


def _gru_math(agg2_ref, hid_ref, b_conv,
              Wri, Wzi, Wni, Wrh, Wzh, Wnh, bri, bzi, bni, brh, bzh, bnh):
    agg = agg2_ref[0] + agg2_ref[1] + b_conv[...]
    x_in = jnp.maximum(agg, 0.0)
    hid = hid_ref[...]

    def mm(a, b):
        return jnp.dot(a, b[...], preferred_element_type=_f32)

    r = jax.nn.sigmoid(mm(x_in, Wri) + bri[...] + mm(hid, Wrh) + brh[...])
    z = jax.nn.sigmoid(mm(x_in, Wzi) + bzi[...] + mm(hid, Wzh) + bzh[...])
    n = jnp.tanh(mm(x_in, Wni) + bni[...] + r * (mm(hid, Wnh) + bnh[...]))
    return (1.0 - z) * n + z * hid


def _gru_y_body(agg2_ref, hid_ref, b_conv, Wri, Wzi, Wni, Wrh, Wzh, Wnh,
                bri, bzi, bni, brh, bzh, bnh, w2d_ref, hnew_ref, y_ref):
    hnew = _gru_math(agg2_ref, hid_ref, b_conv, Wri, Wzi, Wni, Wrh, Wzh, Wnh,
                     bri, bzi, bni, brh, bzh, bnh)
    hnew_ref[...] = hnew
    y_ref[...] = jnp.dot(hnew, w2d_ref[...], preferred_element_type=_f32)


def _gru_last_body(agg2_ref, hid_ref, b_conv, Wri, Wzi, Wni, Wrh, Wzh, Wnh,
                   bri, bzi, bni, brh, bzh, bnh, hnew_ref):
    hnew_ref[...] = _gru_math(agg2_ref, hid_ref, b_conv, Wri, Wzi, Wni,
                              Wrh, Wzh, Wnh, bri, bzi, bni, brh, bzh, bnh)


def _gru_step(agg2, hid, w2d, b_conv2d, gw, with_y):
    wspec = [pl.BlockSpec((D, D), lambda i: (0, 0))] * 6
    bspec = [pl.BlockSpec((1, D), lambda i: (0, 0))] * 6
    in_specs = ([pl.BlockSpec((NC, _RB, D), lambda i: (0, i, 0)),
                 pl.BlockSpec((_RB, D), lambda i: (i, 0)),
                 pl.BlockSpec((1, D), lambda i: (0, 0))]
                + wspec + bspec)
    args = [agg2, hid, b_conv2d] + gw
    if with_y:
        in_specs.append(pl.BlockSpec((D, NCLS * D), lambda i: (0, 0)))
        return pl.pallas_call(
            _gru_y_body,
            grid=(N // _RB,),
            in_specs=in_specs,
            out_specs=(pl.BlockSpec((_RB, D), lambda i: (i, 0)),
                       pl.BlockSpec((_RB, NCLS * D), lambda i: (i, 0))),
            out_shape=(jax.ShapeDtypeStruct((N, D), _f32),
                       jax.ShapeDtypeStruct((N, NCLS * D), _f32)),
        )(*args, w2d)
    return pl.pallas_call(
        _gru_last_body,
        grid=(N // _RB,),
        in_specs=in_specs,
        out_specs=pl.BlockSpec((_RB, D), lambda i: (i, 0)),
        out_shape=jax.ShapeDtypeStruct((N, D), _f32),
    )(*args)


def _tail_body(x_ref, Wi_q, Wi_r, Wf_q, Wf_r, Wg_q, Wg_r, Wo_q, Wo_r,
               Whi, Whf, Whg, Who, bi, bf, bg, bo,
               Wp1q, Wp1r, bp1, Wp2, bp2, out_ref):
    x = x_ref[...]

    def mm(a, b):
        return jnp.dot(a, b[...], preferred_element_type=_f32)

    hc = jnp.zeros((1, D), _f32)
    cc = jnp.zeros((1, D), _f32)
    q = jnp.zeros((1, D), _f32)
    readout = jnp.zeros((1, D), _f32)
    for _ in range(3):
        i_g = jax.nn.sigmoid(mm(q, Wi_q) + mm(readout, Wi_r) + mm(hc, Whi)
                             + bi[...])
        f_g = jax.nn.sigmoid(mm(q, Wf_q) + mm(readout, Wf_r) + mm(hc, Whf)
                             + bf[...])
        g_g = jnp.tanh(mm(q, Wg_q) + mm(readout, Wg_r) + mm(hc, Whg)
                       + bg[...])
        o_g = jax.nn.sigmoid(mm(q, Wo_q) + mm(readout, Wo_r) + mm(hc, Who)
                             + bo[...])
        cc = f_g * cc + i_g * g_g
        hc = o_g * jnp.tanh(cc)
        q = hc
        en = jnp.sum(x * q, axis=-1, keepdims=True)
        mx = jnp.max(en)
        w = jnp.exp(en - mx)
        readout = jnp.sum(x * w, axis=0, keepdims=True) / jnp.sum(w)

    hmlp = jnp.maximum(mm(q, Wp1q) + mm(readout, Wp1r) + bp1[...], 0.0)
    out_ref[...] = mm(hmlp, Wp2) + bp2[...]


def _tail(x, tw):
    return pl.pallas_call(
        _tail_body,
        out_shape=jax.ShapeDtypeStruct((1, D), _f32),
    )(x, *tw)


def kernel(edge_index, h, e, Eh0, Eh1, Eh2, Ee0, Ee1, W_proj, b_proj,
           W_e1, b_e1, W_e2, b_e2, b_conv,
           W_ih_gru, W_hh_gru, b_ih_gru, b_hh_gru,
           W_ih_lstm, W_hh_lstm, b_ih_lstm, b_hh_lstm,
           W_p1, b_p1, W_p2, b_p2):
    srcb = edge_index[0].reshape(NW, NCHUNK, CHUNK).astype(jnp.int32)
    dstb = edge_index[1].reshape(NW, NCHUNK, CHUNK).astype(jnp.int32)
    e0b = e[:, 0].reshape(NW, NCHUNK, CHUNK).astype(jnp.int32)
    e1b = e[:, 1].reshape(NW, NCHUNK, CHUNK).astype(jnp.int32)
    h0 = h[:, 0:1].astype(jnp.int32)
    h1 = h[:, 1:2].astype(jnp.int32)
    h2 = h[:, 2:3].astype(jnp.int32)
    We2t = W_e2.reshape(2 * D, D, D).transpose(1, 0, 2)
    be2r = b_e2.reshape(D, D)
    Wgi = W_ih_gru.T
    Wgh = W_hh_gru.T
    gw = [Wgi[:, 0:D], Wgi[:, D:2 * D], Wgi[:, 2 * D:3 * D],
          Wgh[:, 0:D], Wgh[:, D:2 * D], Wgh[:, 2 * D:3 * D],
          b_ih_gru[0:D].reshape(1, D), b_ih_gru[D:2 * D].reshape(1, D),
          b_ih_gru[2 * D:3 * D].reshape(1, D),
          b_hh_gru[0:D].reshape(1, D), b_hh_gru[D:2 * D].reshape(1, D),
          b_hh_gru[2 * D:3 * D].reshape(1, D)]
    Wli = W_ih_lstm.T
    Wlh = W_hh_lstm.T
    bl_i = b_ih_lstm.reshape(1, 4 * D)
    bl_h = b_hh_lstm.reshape(1, 4 * D)
    tw = []
    for g in range(4):
        tw += [Wli[0:D, g * D:(g + 1) * D], Wli[D:2 * D, g * D:(g + 1) * D]]
    tw += [Wlh[:, g * D:(g + 1) * D] for g in range(4)]
    tw += [(bl_i[:, g * D:(g + 1) * D] + bl_h[:, g * D:(g + 1) * D])
           for g in range(4)]
    tw += [W_p1[0:D], W_p1[D:2 * D], b_p1.reshape(1, D), W_p2,
           b_p2.reshape(1, D)]

    x0, wall = _prep(h0, h1, h2, Eh0, Eh1, Eh2,
                     W_proj[0:8], W_proj[8:12], W_proj[12:16],
                     b_proj.reshape(1, D), Ee0, Ee1,
                     W_e1[0:4], W_e1[4:8], b_e1.reshape(1, 2 * D), We2t, be2r)
    w2d = wall.reshape(D, NCLS * D)
    y = _ymm(x0, w2d)
    hid = x0
    for t in range(3):
        agg2 = _sc_step(y.reshape(N * NCLS, D), srcb, e0b, e1b, dstb)
        agg2 = agg2.reshape(NC, N_PAD, D)[:, :N, :]
        if t < 2:
            hid, y = _gru_step(agg2, hid, w2d, b_conv.reshape(1, D), gw,
                               with_y=True)
        else:
            hid = _gru_step(agg2, hid, None, b_conv.reshape(1, D), gw,
                            with_y=False)
    return _tail(hid, tw)
